# Initial kernel scaffold; baseline (speedup 1.0000x reference)
import jax
import jax.numpy as jnp
from jax.experimental import pallas as pl


def kernel(input_ids, scores_ids, words_table, scores_table):
    raise NotImplementedError("write your pallas kernel here")


# speedup vs baseline: 2.1327x; 2.1327x over previous
import functools

import jax
import jax.numpy as jnp
from jax import lax
from jax.experimental import pallas as pl
from jax.experimental.pallas import tpu as pltpu, tpu_sc as plsc

BATCH = 4096
SEQ = 50
EMBED_DIM = 64
N = BATCH * SEQ

NUM_CORES = 2
NUM_SUBCORES = 16
NUM_WORKERS = NUM_CORES * NUM_SUBCORES
PER_WORKER = N // NUM_WORKERS
CHUNK = 128
NUM_CHUNKS = PER_WORKER // CHUNK
LANES = 16


def _emb_sum_kernel(iw_hbm, is_hbm, words_hbm, scores_hbm, out_hbm,
                    idxw_v, idxs_v, wbuf, sbuf, semw, sems):
    wid = lax.axis_index("s") * NUM_CORES + lax.axis_index("c")
    base = wid * PER_WORKER
    pltpu.sync_copy(iw_hbm.at[pl.ds(base, PER_WORKER)], idxw_v)
    pltpu.sync_copy(is_hbm.at[pl.ds(base, PER_WORKER)], idxs_v)

    def chunk_body(c, carry):
        off = c * CHUNK
        cpw = pltpu.async_copy(
            words_hbm.at[idxw_v.at[pl.ds(off, CHUNK)]], wbuf, semw)
        cps = pltpu.async_copy(
            scores_hbm.at[idxs_v.at[pl.ds(off, CHUNK)]], sbuf, sems)
        cpw.wait()
        cps.wait()

        def add_row(r, carry2):
            for j in range(EMBED_DIM // LANES):
                sl = pl.ds(j * LANES, LANES)
                wbuf[r, sl] = wbuf[r, sl] + sbuf[r, sl]
            return carry2

        lax.fori_loop(0, CHUNK, add_row, 0, unroll=4)
        pltpu.sync_copy(wbuf, out_hbm.at[pl.ds(base + off, CHUNK)])
        return carry

    lax.fori_loop(0, NUM_CHUNKS, chunk_body, 0)


@jax.jit
def kernel(input_ids, scores_ids, words_table, scores_table):
    iw = input_ids.reshape(-1).astype(jnp.int32)
    isc = scores_ids.reshape(-1).astype(jnp.int32)
    mesh = plsc.VectorSubcoreMesh(core_axis_name="c", subcore_axis_name="s")
    run = functools.partial(
        pl.kernel,
        mesh=mesh,
        compiler_params=pltpu.CompilerParams(use_tc_tiling_on_sc=False),
        out_type=jax.ShapeDtypeStruct((N, EMBED_DIM), jnp.float32),
        scratch_types=[
            pltpu.VMEM((PER_WORKER,), jnp.int32),
            pltpu.VMEM((PER_WORKER,), jnp.int32),
            pltpu.VMEM((CHUNK, EMBED_DIM), jnp.float32),
            pltpu.VMEM((CHUNK, EMBED_DIM), jnp.float32),
            pltpu.SemaphoreType.DMA,
            pltpu.SemaphoreType.DMA,
        ],
    )(_emb_sum_kernel)
    out = run(iw, isc, words_table, scores_table)
    return out.reshape(BATCH, SEQ, EMBED_DIM)
